# baseline (speedup 1.0000x reference)
import functools

import jax
import jax.numpy as jnp
from jax import lax
from jax.experimental import pallas as pl
from jax.experimental.pallas import tpu as pltpu
from jax.experimental.pallas import tpu_sc as plsc


CHUNK = 128


@functools.lru_cache(maxsize=None)
def _build(B, D, NC, NS):
    NW = NC * NS
    per_w = B // NW
    n_chunks = per_w // CHUNK
    assert per_w % CHUNK == 0

    mesh = plsc.VectorSubcoreMesh(
        core_axis_name="c", subcore_axis_name="s",
        num_cores=NC, num_subcores=NS,
    )

    @functools.partial(
        pl.kernel,
        out_type=jax.ShapeDtypeStruct((B, D), jnp.float32),
        mesh=mesh,
        scratch_types=[
            pltpu.VMEM((CHUNK,), jnp.int32),
            pltpu.VMEM((CHUNK, D), jnp.float32),
            pltpu.SemaphoreType.DMA,
        ],
        compiler_params=pltpu.CompilerParams(use_tc_tiling_on_sc=False),
    )
    def gather_kernel(idx_hbm, table_hbm, out_hbm, idx_v, rows_v, sem):
        wid = lax.axis_index("s") * NC + lax.axis_index("c")
        base = wid * per_w

        def body(i, carry):
            off = base + i * CHUNK
            pltpu.sync_copy(idx_hbm.at[pl.ds(off, CHUNK)], idx_v)
            pltpu.async_copy(table_hbm.at[idx_v], rows_v, sem).wait()
            pltpu.sync_copy(rows_v, out_hbm.at[pl.ds(off, CHUNK)])
            return carry

        lax.fori_loop(0, n_chunks, body, 0)

    return gather_kernel


def kernel(token_ids, weight):
    B0, S = token_ids.shape
    V, D = weight.shape
    B = B0 * S
    info = plsc.get_sparse_core_info()
    f = _build(B, D, info.num_cores, info.num_subcores)
    flat_idx = token_ids.reshape(B).astype(jnp.int32)
    out = f(flat_idx, weight)
    return out.reshape(B0, S, D)


# speedup vs baseline: 1.1939x; 1.1939x over previous
import functools

import jax
import jax.numpy as jnp
from jax import lax
from jax.experimental import pallas as pl
from jax.experimental.pallas import tpu as pltpu
from jax.experimental.pallas import tpu_sc as plsc


CHUNK = 128
K = 5


@functools.lru_cache(maxsize=None)
def _build(B, D, NC, NS):
    NW = NC * NS
    per_w = B // NW
    n_chunks = per_w // CHUNK
    n_pass = n_chunks // K
    assert B % NW == 0 and per_w % (CHUNK * K) == 0 and n_pass % 2 == 0

    idx_bytes = K * CHUNK * 4
    rows_bytes = K * CHUNK * D * 4

    mesh = plsc.VectorSubcoreMesh(
        core_axis_name="c", subcore_axis_name="s",
        num_cores=NC, num_subcores=NS,
    )

    @functools.partial(
        pl.kernel,
        out_type=jax.ShapeDtypeStruct((B, D), jnp.float32),
        mesh=mesh,
        scratch_types=[
            pltpu.VMEM((2, K, CHUNK), jnp.int32),
            pltpu.VMEM((2, K, CHUNK, D), jnp.float32),
            pltpu.SemaphoreType.DMA,
            pltpu.SemaphoreType.DMA,
            pltpu.SemaphoreType.DMA,
            pltpu.SemaphoreType.DMA,
        ],
        compiler_params=pltpu.CompilerParams(use_tc_tiling_on_sc=False),
    )
    def gather_kernel(idx_hbm, table_hbm, out_hbm, idx_v, rows_v,
                      isem, gsem, ssem0, ssem1):
        wid = lax.axis_index("s") * NC + lax.axis_index("c")
        cbase = wid * n_chunks
        ssems = (ssem0, ssem1)

        def idx_fetch(g, p):
            pltpu.async_copy(
                idx_hbm.at[pl.ds(cbase + g * K, K)], idx_v.at[p], isem)

        def one_pass(go, q):
            g = 2 * go + q

            pltpu.make_async_copy(
                idx_hbm.at[pl.ds(0, K)], idx_v.at[q], isem).wait()

            @pl.when(go >= 1)
            def _():
                for b in range(K):
                    pltpu.make_async_copy(
                        rows_v.at[q, b],
                        out_hbm.at[pl.ds(0, CHUNK)], ssems[q]).wait()

            for b in range(K):
                pltpu.async_copy(
                    table_hbm.at[idx_v.at[q, b]], rows_v.at[q, b], gsem)

            @pl.when(g + 1 < n_pass)
            def _():
                idx_fetch(g + 1, 1 - q)

            for b in range(K):
                pltpu.make_async_copy(
                    table_hbm.at[idx_v.at[q, b]], rows_v.at[q, b],
                    gsem).wait()
            for b in range(K):
                off = cbase * CHUNK + (g * K + b) * CHUNK
                pltpu.async_copy(
                    rows_v.at[q, b], out_hbm.at[pl.ds(off, CHUNK)],
                    ssems[q])

        idx_fetch(0, 0)

        def outer(go, carry):
            one_pass(go, 0)
            one_pass(go, 1)
            return carry

        lax.fori_loop(0, n_pass // 2, outer, 0)

        for q in range(2):
            for b in range(K):
                pltpu.make_async_copy(
                    rows_v.at[q, b],
                    out_hbm.at[pl.ds(0, CHUNK)], ssems[q]).wait()

    return gather_kernel


def kernel(token_ids, weight):
    B0, S = token_ids.shape
    V, D = weight.shape
    B = B0 * S
    info = plsc.get_sparse_core_info()
    f = _build(B, D, info.num_cores, info.num_subcores)
    idx2d = token_ids.reshape(B // CHUNK, CHUNK).astype(jnp.int32)
    out = f(idx2d, weight)
    return out.reshape(B0, S, D)
